# Initial kernel scaffold; baseline (speedup 1.0000x reference)
import jax
import jax.numpy as jnp
from jax.experimental import pallas as pl


def kernel(locations, features):
    raise NotImplementedError("write your pallas kernel here")


# speedup vs baseline: 7.4260x; 7.4260x over previous
import functools

import jax
import jax.numpy as jnp
from jax.experimental import pallas as pl
from jax.experimental.pallas import tpu as pltpu
from jax.experimental.pallas import tpu_sc as plsc

_B = 4
_N = 65536
_C = 128
_M = 16
_LANES = 128
_ROWS = _N // _LANES


def _fps_body(loc_ref, idx_ref, pts_ref):
    xs = loc_ref[:, 0]
    ys = loc_ref[:, 1]
    zs = loc_ref[:, 2]
    shape = (_B, _ROWS, _LANES)
    row = jax.lax.broadcasted_iota(jnp.int32, shape, 1)
    col = jax.lax.broadcasted_iota(jnp.int32, shape, 2)
    lin = row * _LANES + col
    big = jnp.int32(jnp.iinfo(jnp.int32).max)

    def extract(mask, v):
        return jnp.sum(jnp.where(mask, v, 0.0), axis=(1, 2)).reshape(_B, 1, 1)

    dists = jnp.full(shape, 1e10, dtype=jnp.float32)
    cur = jnp.zeros((_B, 1, 1), dtype=jnp.int32)
    idx_ref[:, 0:1] = jnp.zeros((_B, 1), jnp.int32)

    for i in range(1, _M):
        mask = lin == cur
        lx = extract(mask, xs)
        ly = extract(mask, ys)
        lz = extract(mask, zs)
        pts_ref[:, 0:1, i - 1:i] = lx
        pts_ref[:, 1:2, i - 1:i] = ly
        pts_ref[:, 2:3, i - 1:i] = lz
        dx = xs - lx
        dy = ys - ly
        dz = zs - lz
        d = (dx * dx + dy * dy) + dz * dz
        dists = jnp.minimum(dists, d)
        m = jnp.max(dists, axis=(1, 2)).reshape(_B, 1, 1)
        nxt = jnp.min(jnp.where(dists == m, lin, big), axis=(1, 2))
        nxt = nxt.astype(jnp.int32).reshape(_B, 1, 1)
        idx_ref[:, i:i + 1] = nxt.reshape(_B, 1)
        cur = nxt

    mask = lin == cur
    pts_ref[:, 0:1, _M - 1:_M] = extract(mask, xs)
    pts_ref[:, 1:2, _M - 1:_M] = extract(mask, ys)
    pts_ref[:, 2:3, _M - 1:_M] = extract(mask, zs)


_SC_NC = 2
_SC_NS = 16
_SC_L = 16
_GRAN = 16
_NROWS = _N // _GRAN
_PAIRS = _B * _M
_NW = _SC_NC * _SC_NS
_PPW = _PAIRS // _NW


def _sc_gather_features(features, anchor_idx):
    table = features.reshape(_B * _C * _NROWS, _GRAN)
    idx_flat = anchor_idx.reshape(_PAIRS)
    mesh = plsc.VectorSubcoreMesh(core_axis_name="c", subcore_axis_name="s")

    @functools.partial(
        pl.kernel,
        out_type=jax.ShapeDtypeStruct((_B * _M * _C,), jnp.float32),
        mesh=mesh,
        scratch_types=[
            pltpu.VMEM((_PAIRS,), jnp.int32),
            pltpu.VMEM((_PPW, _C), jnp.int32),
            pltpu.VMEM((_C, _GRAN), jnp.float32),
            pltpu.VMEM((_C,), jnp.float32),
            pltpu.SemaphoreType.DMA,
        ],
        compiler_params=pltpu.CompilerParams(
            needs_layout_passes=False, use_tc_tiling_on_sc=False
        ),
    )
    def gather_kernel(t_hbm, i_hbm, o_hbm, idx_v, rows_idx, rows_v, outb, sem):
        wid = jax.lax.axis_index("s") * _SC_NC + jax.lax.axis_index("c")
        pltpu.sync_copy(i_hbm, idx_v)
        lane_iota = jax.lax.iota(jnp.int32, _SC_L)
        for pair_local in range(_PPW):
            p = wid * _PPW + pair_local
            b = p // _M
            idxval = plsc.load_gather(idx_v, [jnp.full((_SC_L,), p, jnp.int32)])
            rowbase = idxval // _GRAN
            lane = jnp.bitwise_and(idxval, _GRAN - 1)
            for j in range(_C // _SC_L):
                c_vec = j * _SC_L + lane_iota
                rows = (b * _C + c_vec) * _NROWS + rowbase
                rows_idx[pair_local, pl.ds(j * _SC_L, _SC_L)] = rows
            pltpu.async_copy(
                t_hbm.at[rows_idx.at[pair_local]], rows_v, sem
            ).wait()
            for j in range(_C // _SC_L):
                vals = plsc.load_gather(rows_v, [j * _SC_L + lane_iota, lane])
                outb[pl.ds(j * _SC_L, _SC_L)] = vals
            off = pl.multiple_of(p * _C, _C)
            pltpu.sync_copy(outb, o_hbm.at[pl.ds(off, _C)])

    out = gather_kernel(table, idx_flat)
    return out.reshape(_B, _M, _C).transpose(0, 2, 1)


def kernel(locations, features):
    loc4 = locations.transpose(0, 2, 1).reshape(_B, 3, _ROWS, _LANES)
    anchor_idx, pts = pl.pallas_call(
        _fps_body,
        out_shape=(
            jax.ShapeDtypeStruct((_B, _M), jnp.int32),
            jax.ShapeDtypeStruct((_B, 3, _M), jnp.float32),
        ),
    )(loc4)
    anchor_points = pts.transpose(0, 2, 1)
    anchor_features = _sc_gather_features(features, anchor_idx)
    return anchor_points, anchor_features, anchor_idx


# speedup vs baseline: 18.1764x; 2.4477x over previous
import functools

import jax
import jax.numpy as jnp
from jax.experimental import pallas as pl
from jax.experimental.pallas import tpu as pltpu
from jax.experimental.pallas import tpu_sc as plsc

_B = 4
_N = 65536
_C = 128
_M = 16
_LANES = 128
_ROWS = _N // _LANES


def _fps_body(loc_ref, idx_ref, pts_ref):
    xs = loc_ref[:, 0]
    ys = loc_ref[:, 1]
    zs = loc_ref[:, 2]
    shape = (_B, _ROWS, _LANES)
    row = jax.lax.broadcasted_iota(jnp.int32, shape, 1)
    col = jax.lax.broadcasted_iota(jnp.int32, shape, 2)
    lin = row * _LANES + col
    big = jnp.int32(jnp.iinfo(jnp.int32).max)

    def extract(mask, v):
        return jnp.sum(jnp.where(mask, v, 0.0), axis=(1, 2)).reshape(_B, 1, 1)

    dists = jnp.full(shape, 1e10, dtype=jnp.float32)
    cur = jnp.zeros((_B, 1, 1), dtype=jnp.int32)
    idx_ref[:, 0:1] = jnp.zeros((_B, 1), jnp.int32)

    for i in range(1, _M):
        mask = lin == cur
        lx = extract(mask, xs)
        ly = extract(mask, ys)
        lz = extract(mask, zs)
        pts_ref[:, 0:1, i - 1:i] = lx
        pts_ref[:, 1:2, i - 1:i] = ly
        pts_ref[:, 2:3, i - 1:i] = lz
        dx = xs - lx
        dy = ys - ly
        dz = zs - lz
        d = (dx * dx + dy * dy) + dz * dz
        dists = jnp.minimum(dists, d)
        m = jnp.max(dists, axis=(1, 2)).reshape(_B, 1, 1)
        nxt = jnp.min(jnp.where(dists == m, lin, big), axis=(1, 2))
        nxt = nxt.astype(jnp.int32).reshape(_B, 1, 1)
        idx_ref[:, i:i + 1] = nxt.reshape(_B, 1)
        cur = nxt

    mask = lin == cur
    pts_ref[:, 0:1, _M - 1:_M] = extract(mask, xs)
    pts_ref[:, 1:2, _M - 1:_M] = extract(mask, ys)
    pts_ref[:, 2:3, _M - 1:_M] = extract(mask, zs)


_SC_NC = 2
_SC_NS = 16
_SC_L = 16
_GRAN = 16
_NROWS = _N // _GRAN
_PAIRS = _B * _M
_NW = _SC_NC * _SC_NS
_PPW = _PAIRS // _NW


def _sc_gather_features(features, anchor_idx):
    table = features.reshape(_B * _C, _N)
    idx_flat = anchor_idx.reshape(_PAIRS)
    mesh = plsc.VectorSubcoreMesh(core_axis_name="c", subcore_axis_name="s")

    @functools.partial(
        pl.kernel,
        out_type=jax.ShapeDtypeStruct((_B * _M * _C,), jnp.float32),
        mesh=mesh,
        scratch_types=[
            pltpu.VMEM((_PAIRS,), jnp.int32),
            pltpu.VMEM((_C, 128), jnp.float32),
            pltpu.VMEM((_C,), jnp.float32),
            pltpu.SemaphoreType.DMA,
        ],
        compiler_params=pltpu.CompilerParams(needs_layout_passes=False),
    )
    def gather_kernel(t_hbm, i_hbm, o_hbm, idx_v, blk_v, outb, sem):
        wid = jax.lax.axis_index("s") * _SC_NC + jax.lax.axis_index("c")
        pltpu.sync_copy(i_hbm, idx_v)
        lane_iota = jax.lax.iota(jnp.int32, _SC_L)
        for pair_local in range(_PPW):
            p = wid * _PPW + pair_local
            b = p // _M
            idxval = plsc.load_gather(idx_v, [jnp.full((_SC_L,), p, jnp.int32)])
            lane = jnp.bitwise_and(idxval, 127)
            idx_s = jnp.max(idxval)
            col0 = pl.multiple_of((idx_s >> 7) << 7, 128)
            row0 = pl.multiple_of(b * _C, _C)
            pltpu.async_copy(
                t_hbm.at[pl.ds(row0, _C), pl.ds(col0, 128)], blk_v, sem
            ).wait()
            for j in range(_C // _SC_L):
                vals = plsc.load_gather(blk_v, [j * _SC_L + lane_iota, lane])
                outb[pl.ds(j * _SC_L, _SC_L)] = vals
            off = pl.multiple_of(p * _C, _C)
            pltpu.sync_copy(outb, o_hbm.at[pl.ds(off, _C)])

    out = gather_kernel(table, idx_flat)
    return out.reshape(_B, _M, _C).transpose(0, 2, 1)


def kernel(locations, features):
    loc4 = locations.transpose(0, 2, 1).reshape(_B, 3, _ROWS, _LANES)
    anchor_idx, pts = pl.pallas_call(
        _fps_body,
        out_shape=(
            jax.ShapeDtypeStruct((_B, _M), jnp.int32),
            jax.ShapeDtypeStruct((_B, 3, _M), jnp.float32),
        ),
    )(loc4)
    anchor_points = pts.transpose(0, 2, 1)
    anchor_features = _sc_gather_features(features, anchor_idx)
    return anchor_points, anchor_features, anchor_idx
